# baseline (speedup 1.0000x reference)
import functools

import jax
import jax.numpy as jnp
from jax import lax
from jax.experimental import pallas as pl
from jax.experimental.pallas import tpu as pltpu
from jax.experimental.pallas import tpu_sc as plsc

_DIMS = (5, 6, 2, 2)
_TROWS = 128
_TCOLS = 128


def _tab_body(w0, w1, w2, w3, o):
    d = w0.shape[1]
    r = lax.broadcasted_iota(jnp.int32, (_TROWS, d), 0)
    digits = (r // 24, (r % 24) // 4, (r // 2) % 2, r % 2)
    acc = jnp.zeros((_TROWS, d), jnp.float32)
    for dig, w, n in zip(digits, (w0, w1, w2, w3), _DIMS):
        for k in range(n):
            acc = acc + jnp.where(dig == k, 1.0, 0.0) * w[k]
    o[...] = jnp.concatenate(
        [acc, jnp.zeros((_TROWS, _TCOLS - d), jnp.float32)], axis=1)


def _build_table(W0, W1, W2, W3):
    return pl.pallas_call(
        _tab_body,
        out_shape=jax.ShapeDtypeStruct((_TROWS, _TCOLS), jnp.float32),
    )(W0, W1, W2, W3)


def _sc_gather(pk, ftab, E, D):
    info = plsc.get_sparse_core_info()
    NW = info.num_cores * info.num_subcores
    B = 400
    NG = B // 16
    SL = 80
    NSTR = B // SL
    nblk = E // B
    assert E % B == 0

    mesh = plsc.VectorSubcoreMesh(core_axis_name="c", subcore_axis_name="s")

    @functools.partial(
        pl.kernel,
        mesh=mesh,
        out_type=jax.ShapeDtypeStruct((E, _TCOLS), jnp.float32),
        scratch_types=[
            (pltpu.VMEM((B,), jnp.int32),) * 2,
            (pltpu.VMEM((B,), jnp.int32),) * 2,
            (pltpu.VMEM((B, _TCOLS), jnp.float32),) * 2,
            pltpu.VMEM_SHARED((_TROWS, _TCOLS), jnp.float32),
            pltpu.SemaphoreType.DMA,
            (pltpu.SemaphoreType.DMA, pltpu.SemaphoreType.DMA),
        ],
    )
    def run(pk_hbm, tab_hbm, out_hbm, pkv, idxv, rows, tab_sh, gsem, osems):
        sid = lax.axis_index("s")
        wid = sid * info.num_cores + lax.axis_index("c")

        @pl.when(sid == 0)
        def _():
            pltpu.sync_copy(tab_hbm, tab_sh)
        plsc.subcore_barrier()

        nblk_w = (nblk - wid + NW - 1) // NW

        def do_block(j, p, osem):
            off = (wid + j * NW) * B
            pltpu.sync_copy(pk_hbm.at[pl.ds(off, B)], pkv[p])
            for g in range(NG):
                v = pkv[p][pl.ds(g * 16, 16)]
                idx = ((v & 0xFF) * 24 + ((v >> 8) & 0xFF) * 4
                       + ((v >> 16) & 0xFF) * 2 + ((v >> 24) & 0xFF))
                idxv[p][pl.ds(g * 16, 16)] = jnp.minimum(idx, _TROWS - 1)
            handles = [
                pltpu.async_copy(
                    tab_sh.at[idxv[p].at[pl.ds(r * SL, SL)]],
                    rows[p].at[pl.ds(r * SL, SL)],
                    gsem,
                )
                for r in range(NSTR)
            ]
            for h in handles:
                h.wait()
            return pltpu.async_copy(rows[p], out_hbm.at[pl.ds(off, B)], osem)

        def block(j, carry):
            for p in (0, 1):

                @pl.when(j % 2 == p)
                def _():
                    @pl.when(j >= 2)
                    def _():
                        pltpu.make_async_copy(
                            rows[p], out_hbm.at[pl.ds(0, B)], osems[p]
                        ).wait()
                    do_block(j, p, osems[p])
            return carry

        lax.fori_loop(0, nblk_w, block, 0)
        for p in (0, 1):

            @pl.when(nblk_w >= p + 1)
            def _():
                pltpu.make_async_copy(
                    rows[p], out_hbm.at[pl.ds(0, B)], osems[p]
                ).wait()

    return run(pk, ftab)


def kernel(edge_attr, W0, W1, W2, W3):
    E = edge_attr.shape[0]
    D = W0.shape[1]
    H = E // 2
    ftab = _build_table(W0, W1, W2, W3)
    pkA = lax.bitcast_convert_type(edge_attr[:H].astype(jnp.uint8), jnp.int32)
    pkB = lax.bitcast_convert_type(edge_attr[H:].astype(jnp.uint8), jnp.int32)
    outA = _sc_gather(pkA, ftab, H, D)
    outB = _sc_gather(pkB, ftab, E - H, D)
    return jnp.concatenate([outA[:, :D], outB[:, :D]], axis=0)


# speedup vs baseline: 1.6116x; 1.6116x over previous
import functools

import jax
import jax.numpy as jnp
from jax import lax
from jax.experimental import pallas as pl
from jax.experimental.pallas import tpu as pltpu
from jax.experimental.pallas import tpu_sc as plsc

_DIMS = (5, 6, 2, 2)
_TROWS = 128
_TCOLS = 128


def _tab_body(w0, w1, w2, w3, o):
    d = w0.shape[1]
    r = lax.broadcasted_iota(jnp.int32, (_TROWS, d), 0)
    digits = (r // 24, (r % 24) // 4, (r // 2) % 2, r % 2)
    acc = jnp.zeros((_TROWS, d), jnp.float32)
    for dig, w, n in zip(digits, (w0, w1, w2, w3), _DIMS):
        for k in range(n):
            acc = acc + jnp.where(dig == k, 1.0, 0.0) * w[k]
    o[...] = jnp.concatenate(
        [acc, jnp.zeros((_TROWS, _TCOLS - d), jnp.float32)], axis=1)


def _build_table(W0, W1, W2, W3):
    return pl.pallas_call(
        _tab_body,
        out_shape=jax.ShapeDtypeStruct((_TROWS, _TCOLS), jnp.float32),
    )(W0, W1, W2, W3)


def _sc_gather(pk, ftab, E, D):
    info = plsc.get_sparse_core_info()
    NW = info.num_cores * info.num_subcores
    B = 400
    NG = B // 16
    SL = 80
    NSTR = B // SL
    nblk = E // B
    assert E % B == 0

    mesh = plsc.VectorSubcoreMesh(core_axis_name="c", subcore_axis_name="s")

    @functools.partial(
        pl.kernel,
        mesh=mesh,
        out_type=jax.ShapeDtypeStruct((E, _TCOLS), jnp.float32),
        scratch_types=[
            (pltpu.VMEM((B,), jnp.int32),) * 2,
            (pltpu.VMEM((B,), jnp.int32),) * 2,
            (pltpu.VMEM((B, _TCOLS), jnp.float32),) * 2,
            pltpu.VMEM_SHARED((_TROWS, _TCOLS), jnp.float32),
            pltpu.SemaphoreType.DMA,
            (pltpu.SemaphoreType.DMA, pltpu.SemaphoreType.DMA),
        ],
    )
    def run(pk_hbm, tab_hbm, out_hbm, pkv, idxv, rows, tab_sh, gsem, osems):
        sid = lax.axis_index("s")
        wid = sid * info.num_cores + lax.axis_index("c")

        @pl.when(sid == 0)
        def _():
            pltpu.sync_copy(tab_hbm, tab_sh)
        plsc.subcore_barrier()

        nblk_w = (nblk - wid + NW - 1) // NW

        def do_block(j, p, osem):
            off = (wid + j * NW) * B
            pltpu.sync_copy(pk_hbm.at[pl.ds(off, B)], pkv[p])
            for g in range(NG):
                v = pkv[p][pl.ds(g * 16, 16)]
                idx = ((v & 0xFF) * 24 + ((v >> 8) & 0xFF) * 4
                       + ((v >> 16) & 0xFF) * 2 + ((v >> 24) & 0xFF))
                idxv[p][pl.ds(g * 16, 16)] = jnp.minimum(idx, _TROWS - 1)
            handles = [
                pltpu.async_copy(
                    tab_sh.at[idxv[p].at[pl.ds(r * SL, SL)]],
                    rows[p].at[pl.ds(r * SL, SL)],
                    gsem,
                )
                for r in range(NSTR)
            ]
            for h in handles:
                h.wait()
            return pltpu.async_copy(rows[p], out_hbm.at[pl.ds(off, B)], osem)

        def block(j, carry):
            for p in (0, 1):

                @pl.when(j % 2 == p)
                def _():
                    @pl.when(j >= 2)
                    def _():
                        pltpu.make_async_copy(
                            rows[p], out_hbm.at[pl.ds(0, B)], osems[p]
                        ).wait()
                    do_block(j, p, osems[p])
            return carry

        lax.fori_loop(0, nblk_w, block, 0)
        for p in (0, 1):

            @pl.when(nblk_w >= p + 1)
            def _():
                pltpu.make_async_copy(
                    rows[p], out_hbm.at[pl.ds(0, B)], osems[p]
                ).wait()

    return run(pk, ftab)


def kernel(edge_attr, W0, W1, W2, W3):
    E = edge_attr.shape[0]
    D = W0.shape[1]
    pk = lax.bitcast_convert_type(edge_attr.astype(jnp.uint8), jnp.int32)
    ftab = _build_table(W0, W1, W2, W3)
    out128 = _sc_gather(pk, ftab, E, D)
    return out128[:, :D]
